# baseline (speedup 1.0000x reference)
import functools

import jax
import jax.numpy as jnp
from jax import lax
from jax.experimental import pallas as pl
from jax.experimental.pallas import tpu as pltpu
from jax.experimental.pallas import tpu_sc as plsc

_W = 16
_WIN_PER_TILE = 128
_LAG = 32


def kernel(x):
    B, S, D = x.shape
    n_win = S - _W + 1
    rows_per_tile = _WIN_PER_TILE + _W - 1

    info = plsc.get_sparse_core_info()
    nc, ns = info.num_cores, info.num_subcores
    n_workers = nc * ns
    lanes_per_batch = n_workers // B

    mesh = plsc.VectorSubcoreMesh(core_axis_name="c", subcore_axis_name="s")

    @functools.partial(
        pl.kernel,
        mesh=mesh,
        out_type=jax.ShapeDtypeStruct((B, n_win, _W, D), x.dtype),
        scratch_types=[
            pltpu.VMEM((rows_per_tile, D), x.dtype),
            pltpu.SemaphoreType.DMA,
            pltpu.SemaphoreType.DMA,
        ],
        compiler_params=pltpu.CompilerParams(use_tc_tiling_on_sc=False),
    )
    def win_align(x_hbm, out_hbm, rows_v, in_sem, out_sem):
        c = lax.axis_index("c")
        s = lax.axis_index("s")
        wid = s * nc + c
        b = wid // lanes_per_batch
        lane = wid % lanes_per_batch
        w0 = jnp.minimum(lane * _WIN_PER_TILE, n_win - _WIN_PER_TILE)

        pltpu.async_copy(
            x_hbm.at[b, pl.ds(w0, rows_per_tile), :], rows_v, in_sem
        ).wait()

        def window_copy(i):
            return pltpu.make_async_copy(
                rows_v.at[pl.ds(i, _W), :],
                out_hbm.at[b, w0 + i, :, :],
                out_sem,
            )

        def body(i, carry):
            window_copy(i).start()

            @pl.when(i >= _LAG)
            def _():
                window_copy(i - _LAG).wait()

            return carry

        lax.fori_loop(0, _WIN_PER_TILE, body, 0)

        def tail(i, carry):
            window_copy(i).wait()
            return carry

        lax.fori_loop(_WIN_PER_TILE - _LAG, _WIN_PER_TILE, tail, 0)

    return win_align(x)
